# baseline (speedup 1.0000x reference)
import functools

import jax
import jax.numpy as jnp
from jax import lax
from jax.experimental import pallas as pl
from jax.experimental.pallas import tpu as pltpu
from jax.experimental.pallas import tpu_sc as plsc

NC, NS = 2, 16
NW = NC * NS
BT = 128
NG = 4
NT = 2
PD = 3


def _emb_kernel(idx_hbm, table_hbm, out_hbm, idx_v, g_v, t_v, gsem, osem,
                *, units, d):
    wid = lax.axis_index("s") * NC + lax.axis_index("c")
    base = wid * units
    ntile = d // 8

    pltpu.sync_copy(idx_hbm.at[pl.ds(base, units)], idx_v)

    def gather_unit(j, b):
        pltpu.async_copy(table_hbm.at[idx_v.at[j]], g_v.at[b], gsem.at[b])

    for b in range(PD):
        gather_unit(b, b)

    def body(j, carry):
        u = base + j
        h = u // 32
        tb = lax.rem(u, 32)

        @pl.when(j + PD < units)
        def _():
            for bs in range(NG):
                @pl.when(lax.rem(j + PD, NG) == bs)
                def _():
                    gather_unit(j + PD, bs)

        for bs in range(NG):
            @pl.when(lax.rem(j, NG) == bs)
            def _():
                pltpu.make_async_copy(
                    table_hbm.at[idx_v.at[0]], g_v.at[bs],
                    gsem.at[bs]).wait()

        for bw in range(NT):
            @pl.when(lax.rem(j, NT) == bw)
            def _():
                @pl.when(j >= NT)
                def _():
                    for tr in range(ntile):
                        pltpu.make_async_copy(
                            t_v.at[bw, tr], out_hbm.at[0, tr, 0],
                            osem.at[bw]).wait()

                for bs in range(NG):
                    @pl.when(lax.rem(j, NG) == bs)
                    def _():
                        g = g_v.at[bs]
                        iota = lax.iota(jnp.int32, 16)
                        rowsets = [iota + (16 * c) for c in range(BT // 16)]
                        colsets = [jnp.full((16,), c, jnp.int32)
                                   for c in range(d)]
                        for tr in range(ntile):
                            for r in range(8):
                                dd = tr * 8 + r
                                for c in range(BT // 16):
                                    t_v[bw, tr, r, pl.ds(16 * c, 16)] = (
                                        plsc.load_gather(
                                            g, [rowsets[c], colsets[dd]]))

                for tr in range(ntile):
                    pltpu.async_copy(t_v.at[bw, tr], out_hbm.at[h, tr, tb],
                                     osem.at[bw])
        return carry

    lax.fori_loop(0, units, body, 0)

    for bw in range(NT):
        for tr in range(ntile):
            pltpu.make_async_copy(t_v.at[bw, tr], out_hbm.at[0, tr, 0],
                                  osem.at[bw]).wait()


def kernel(input, table):
    batch, hist = input.shape
    n_vocab, d = table.shape
    total = batch * hist
    n_units = total // BT
    units = n_units // NW
    assert total == n_units * BT and n_units == units * NW
    assert batch % BT == 0 and d % 8 == 0

    idx = input.T.reshape(n_units, BT).astype(jnp.int32)

    mesh = plsc.VectorSubcoreMesh(core_axis_name="c", subcore_axis_name="s")
    k = functools.partial(
        pl.kernel,
        mesh=mesh,
        compiler_params=pltpu.CompilerParams(use_tc_tiling_on_sc=False,
                                             needs_layout_passes=False),
        out_type=jax.ShapeDtypeStruct((hist, d // 8, batch // BT, 8, BT),
                                      jnp.float32),
        scratch_types=[
            pltpu.VMEM((units, BT), jnp.int32),
            pltpu.VMEM((NG, BT, d), jnp.float32),
            pltpu.VMEM((NT, d // 8, 8, BT), jnp.float32),
            pltpu.SemaphoreType.DMA((NG,)),
            pltpu.SemaphoreType.DMA((NT,)),
        ],
    )(functools.partial(_emb_kernel, units=units, d=d))
    out = k(idx, table)
    return out.transpose(2, 4, 0, 1, 3).reshape(batch, hist, d)


# speedup vs baseline: 1.0585x; 1.0585x over previous
import functools

import jax
import jax.numpy as jnp
from jax import lax
from jax.experimental import pallas as pl
from jax.experimental.pallas import tpu as pltpu
from jax.experimental.pallas import tpu_sc as plsc

NC, NS = 2, 16
NW = NC * NS
BT = 128
NG = 4
NT = 2
PD = 3


def _emb_kernel(idx_hbm, table_hbm, out_hbm, idx_v, g_v, t_v, gsem, osem,
                *, units, d):
    wid = lax.axis_index("s") * NC + lax.axis_index("c")
    base = wid * units
    ntile = d // 8

    pltpu.sync_copy(idx_hbm.at[pl.ds(base, units)], idx_v)

    def gather_unit(j, b):
        pltpu.async_copy(table_hbm.at[idx_v.at[j]], g_v.at[b], gsem.at[b])

    for b in range(PD):
        gather_unit(b, b)

    def body(j, carry):
        u = base + j
        h = u // 32
        tb = lax.rem(u, 32)

        bg = lax.rem(j, NG)
        bw = lax.rem(j, NT)

        @pl.when(j + PD < units)
        def _():
            gather_unit(j + PD, lax.rem(j + PD, NG))

        pltpu.make_async_copy(
            table_hbm.at[idx_v.at[0]], g_v.at[bg], gsem.at[bg]).wait()

        @pl.when(j >= NT)
        def _():
            for tr in range(ntile):
                pltpu.make_async_copy(
                    t_v.at[bw, tr], out_hbm.at[0, tr, 0],
                    osem.at[bw]).wait()

        g = g_v.at[bg]
        t = t_v.at[bw]
        iota = lax.iota(jnp.int32, 16)
        rowsets = [iota + (16 * c) for c in range(BT // 16)]
        colsets = [jnp.full((16,), c, jnp.int32) for c in range(d)]
        for tr in range(ntile):
            for r in range(8):
                dd = tr * 8 + r
                for c in range(BT // 16):
                    t[tr, r, pl.ds(16 * c, 16)] = (
                        plsc.load_gather(g, [rowsets[c], colsets[dd]]))

        for tr in range(ntile):
            pltpu.async_copy(t_v.at[bw, tr], out_hbm.at[h, tr, tb],
                             osem.at[bw])
        return carry

    lax.fori_loop(0, units, body, 0)

    for bw in range(NT):
        for tr in range(ntile):
            pltpu.make_async_copy(t_v.at[bw, tr], out_hbm.at[0, tr, 0],
                                  osem.at[bw]).wait()


def kernel(input, table):
    batch, hist = input.shape
    n_vocab, d = table.shape
    total = batch * hist
    n_units = total // BT
    units = n_units // NW
    assert total == n_units * BT and n_units == units * NW
    assert batch % BT == 0 and d % 8 == 0

    idx = input.T.reshape(n_units, BT).astype(jnp.int32)

    mesh = plsc.VectorSubcoreMesh(core_axis_name="c", subcore_axis_name="s")
    k = functools.partial(
        pl.kernel,
        mesh=mesh,
        compiler_params=pltpu.CompilerParams(use_tc_tiling_on_sc=False,
                                             needs_layout_passes=False),
        out_type=jax.ShapeDtypeStruct((hist, d // 8, batch // BT, 8, BT),
                                      jnp.float32),
        scratch_types=[
            pltpu.VMEM((units, BT), jnp.int32),
            pltpu.VMEM((NG, BT, d), jnp.float32),
            pltpu.VMEM((NT, d // 8, 8, BT), jnp.float32),
            pltpu.SemaphoreType.DMA((NG,)),
            pltpu.SemaphoreType.DMA((NT,)),
        ],
    )(functools.partial(_emb_kernel, units=units, d=d))
    out = k(idx, table)
    return out.transpose(2, 4, 0, 1, 3).reshape(batch, hist, d)


# speedup vs baseline: 1.2063x; 1.1396x over previous
import functools

import jax
import jax.numpy as jnp
from jax import lax
from jax.experimental import pallas as pl
from jax.experimental.pallas import tpu as pltpu
from jax.experimental.pallas import tpu_sc as plsc

NC, NS = 2, 16
NW = NC * NS
GW = 640
NB = 4
PD = 2


def _emb_kernel(idx_hbm, table_hbm, out_hbm, idx_v, rows_v, gsem, osem,
                *, steps):
    wid = lax.axis_index("s") * NC + lax.axis_index("c")
    base = wid * steps

    pltpu.sync_copy(idx_hbm.at[pl.ds(base, steps)], idx_v)

    def gather_group(t, b):
        pltpu.async_copy(table_hbm.at[idx_v.at[t]], rows_v.at[b],
                         gsem.at[b])

    for b in range(PD):
        gather_group(b, b)

    def body(g, carry):
        for bs in range(NB):
            t = g * NB + bs
            tp = t + PD
            bp = (bs + PD) % NB

            @pl.when(tp < steps)
            def _():
                @pl.when(tp >= NB)
                def _():
                    pltpu.make_async_copy(
                        rows_v.at[bp], out_hbm.at[0], osem.at[bp]).wait()

                gather_group(tp, bp)

            pltpu.make_async_copy(
                table_hbm.at[idx_v.at[0]], rows_v.at[bs],
                gsem.at[bs]).wait()
            pltpu.async_copy(rows_v.at[bs], out_hbm.at[base + t],
                             osem.at[bs])
        return carry

    lax.fori_loop(0, steps // NB, body, 0)

    for b in range(NB):
        pltpu.make_async_copy(rows_v.at[b], out_hbm.at[0], osem.at[b]).wait()


def kernel(input, table):
    batch, hist = input.shape
    n_vocab, d = table.shape
    total = batch * hist
    n_grp = total // GW
    steps = n_grp // NW
    assert total == n_grp * GW and n_grp == steps * NW
    assert steps % NB == 0

    idx = input.reshape(n_grp, GW).astype(jnp.int32)

    mesh = plsc.VectorSubcoreMesh(core_axis_name="c", subcore_axis_name="s")
    k = functools.partial(
        pl.kernel,
        mesh=mesh,
        compiler_params=pltpu.CompilerParams(use_tc_tiling_on_sc=False),
        out_type=jax.ShapeDtypeStruct((n_grp, GW, d), jnp.float32),
        scratch_types=[
            pltpu.VMEM((steps, GW), jnp.int32),
            pltpu.VMEM((NB, GW, d), jnp.float32),
            pltpu.SemaphoreType.DMA((NB,)),
            pltpu.SemaphoreType.DMA((NB,)),
        ],
    )(functools.partial(_emb_kernel, steps=steps))
    out = k(idx, table)
    return out.reshape(batch, hist, d)
